# baseline (speedup 1.0000x reference)
import functools

import jax
import jax.numpy as jnp
from jax import lax
from jax.experimental import pallas as pl
from jax.experimental.pallas import tpu as pltpu
from jax.experimental.pallas import tpu_sc as plsc

_PROMPT_LAYER_INDICES = (0,)
_NC, _NS, _L = 2, 16, 16
_NW = _NC * _NS


@functools.cache
def _build_sc_copy(B, T, E, NT, P, zero_prefix):
    PR = B * P
    IR = B * T
    assert PR % _NW == 0 and IR % _NW == 0
    pr_w = PR // _NW
    ir_w = IR // _NW
    assert pr_w == _L
    CH = _L
    n_chunks = ir_w // CH
    assert ir_w % CH == 0
    w_per_b = _NW // B
    assert P == w_per_b * pr_w

    mesh = plsc.VectorSubcoreMesh(core_axis_name="c", subcore_axis_name="s")

    def body(in_hbm, tab_hbm, pidx_hbm, out_hbm,
             idx_v, buf0, buf1, si0, si1, so0, so1):
        wid = lax.axis_index("s") * _NC + lax.axis_index("c")
        w_b = wid // w_per_b
        w_c = wid % w_per_b
        rr = w_c * _L
        if not zero_prefix:
            pltpu.sync_copy(pidx_hbm.at[pl.ds(wid * pr_w, pr_w)], idx_v)
            gather = pltpu.make_async_copy(tab_hbm.at[idx_v], buf0, si0)
            gather.start()
        t0 = w_c * ir_w
        src0 = in_hbm.at[pl.ds(w_b * T + t0, CH)]
        pltpu.make_async_copy(src0, buf1, si1).start()
        if zero_prefix:
            def zcol(j, carry):
                for r in range(_L):
                    buf0[r, pl.ds(j * _L, _L)] = jnp.zeros((_L,), jnp.float32)
                return carry
            lax.fori_loop(0, E // _L, zcol, 0)
        else:
            gather.wait()
        orow0 = w_b * (P + T) + rr
        pltpu.make_async_copy(buf0, out_hbm.at[pl.ds(orow0, CH)], so0).start()
        bufs = (buf0, buf1)
        isems = (si0, si1)
        osems = (so0, so1)
        for k in range(n_chunks):
            slot = (k + 1) % 2
            pltpu.make_async_copy(
                in_hbm.at[pl.ds(w_b * T + t0 + k * CH, CH)],
                bufs[slot], isems[slot]).wait()
            if k + 1 < n_chunks:
                ns = k % 2
                pltpu.make_async_copy(
                    bufs[ns], out_hbm.at[pl.ds(0, CH)], osems[ns]).wait()
                pltpu.make_async_copy(
                    in_hbm.at[pl.ds(w_b * T + t0 + (k + 1) * CH, CH)],
                    bufs[ns], isems[ns]).start()
            dst = out_hbm.at[pl.ds(w_b * (P + T) + P + t0 + k * CH, CH)]
            pltpu.make_async_copy(bufs[slot], dst, osems[slot]).start()
        pltpu.make_async_copy(buf0, out_hbm.at[pl.ds(0, CH)], so0).wait()
        pltpu.make_async_copy(buf1, out_hbm.at[pl.ds(0, CH)], so1).wait()

    return pl.kernel(
        body,
        out_type=jax.ShapeDtypeStruct((B * (P + T), E), jnp.float32),
        mesh=mesh,
        scratch_types=[
            pltpu.VMEM((_L,), jnp.int32),
            pltpu.VMEM((CH, E), jnp.float32),
            pltpu.VMEM((CH, E), jnp.float32),
            pltpu.SemaphoreType.DMA,
            pltpu.SemaphoreType.DMA,
            pltpu.SemaphoreType.DMA,
            pltpu.SemaphoreType.DMA,
        ],
    )


def kernel(input_embedding, layer_idx, task_ids, prompt_table):
    B, T, E = input_embedding.shape
    NT, P, _ = prompt_table.shape
    if P == 0:
        return input_embedding
    active = jnp.any(
        jnp.asarray(_PROMPT_LAYER_INDICES, jnp.int32)
        == jnp.asarray(layer_idx, jnp.int32))
    in_rows = input_embedding.reshape(B * T, E)
    tab_rows = prompt_table.reshape(NT * P, E)
    prow_idx = (task_ids.astype(jnp.int32)[:, None] * P
                + jnp.arange(P, dtype=jnp.int32)[None, :]).reshape(B * P)
    out = lax.cond(
        active,
        lambda a, b, c: _build_sc_copy(B, T, E, NT, P, False)(a, b, c),
        lambda a, b, c: _build_sc_copy(B, T, E, NT, P, True)(a, b, c),
        in_rows, tab_rows, prow_idx)
    return out.reshape(B, P + T, E)


# speedup vs baseline: 1.0068x; 1.0068x over previous
import functools

import jax
import jax.numpy as jnp
from jax import lax
from jax.experimental import pallas as pl
from jax.experimental.pallas import tpu as pltpu
from jax.experimental.pallas import tpu_sc as plsc

_PROMPT_LAYER_INDICES = (0,)
_NC, _NS, _L = 2, 16, 16
_NW = _NC * _NS


@functools.cache
def _build_sc_copy(B, T, E, NT, P, zero_prefix):
    PR = B * P
    IR = B * T
    assert PR % _NW == 0 and IR % _NW == 0
    pr_w = PR // _NW
    ir_w = IR // _NW
    assert pr_w == _L
    CH = _L
    NB = 3
    n_chunks = ir_w // CH
    assert ir_w % CH == 0 and n_chunks >= NB
    w_per_b = _NW // B
    assert P == w_per_b * pr_w

    mesh = plsc.VectorSubcoreMesh(core_axis_name="c", subcore_axis_name="s")

    def body(in_hbm, tab_hbm, pidx_hbm, out_hbm,
             idx_v, b0, b1, b2, si0, si1, si2, so0, so1, so2):
        bufs = (b0, b1, b2)
        isems = (si0, si1, si2)
        osems = (so0, so1, so2)
        wid = lax.axis_index("s") * _NC + lax.axis_index("c")
        w_b = wid // w_per_b
        w_c = wid % w_per_b
        rr = w_c * _L
        t0 = w_c * ir_w

        def in_src(k):
            return in_hbm.at[pl.ds(w_b * T + t0 + k * CH, CH)]

        def out_dst(k):
            return out_hbm.at[pl.ds(w_b * (P + T) + P + t0 + k * CH, CH)]

        if not zero_prefix:
            pltpu.sync_copy(pidx_hbm.at[pl.ds(wid * pr_w, pr_w)], idx_v)
            gather = pltpu.make_async_copy(tab_hbm.at[idx_v], b0, si0)
            gather.start()
        pltpu.make_async_copy(in_src(0), bufs[1], isems[1]).start()
        if zero_prefix:
            def zcol(j, carry):
                for r in range(_L):
                    b0[r, pl.ds(j * _L, _L)] = jnp.zeros((_L,), jnp.float32)
                return carry
            lax.fori_loop(0, E // _L, zcol, 0)
        else:
            gather.wait()
        orow0 = w_b * (P + T) + rr
        pltpu.make_async_copy(b0, out_hbm.at[pl.ds(orow0, pr_w)], so0).start()
        for k in range(n_chunks):
            s = (k + 1) % NB
            pltpu.make_async_copy(in_src(k), bufs[s], isems[s]).wait()
            pltpu.make_async_copy(bufs[s], out_dst(k), osems[s]).start()
            if k + 1 < n_chunks:
                ns = (k + 2) % NB
                if k >= 1:
                    pltpu.make_async_copy(
                        bufs[ns], out_hbm.at[pl.ds(0, CH)], osems[ns]).wait()
                pltpu.make_async_copy(
                    in_src(k + 1), bufs[ns], isems[ns]).start()
        for s in range(NB):
            pltpu.make_async_copy(
                bufs[s], out_hbm.at[pl.ds(0, CH)], osems[s]).wait()

    return pl.kernel(
        body,
        out_type=jax.ShapeDtypeStruct((B * (P + T), E), jnp.float32),
        mesh=mesh,
        scratch_types=[
            pltpu.VMEM((_L,), jnp.int32),
            pltpu.VMEM((CH, E), jnp.float32),
            pltpu.VMEM((CH, E), jnp.float32),
            pltpu.VMEM((CH, E), jnp.float32),
            pltpu.SemaphoreType.DMA,
            pltpu.SemaphoreType.DMA,
            pltpu.SemaphoreType.DMA,
            pltpu.SemaphoreType.DMA,
            pltpu.SemaphoreType.DMA,
            pltpu.SemaphoreType.DMA,
        ],
    )


def kernel(input_embedding, layer_idx, task_ids, prompt_table):
    B, T, E = input_embedding.shape
    NT, P, _ = prompt_table.shape
    if P == 0:
        return input_embedding
    active = jnp.any(
        jnp.asarray(_PROMPT_LAYER_INDICES, jnp.int32)
        == jnp.asarray(layer_idx, jnp.int32))
    in_rows = input_embedding.reshape(B * T, E)
    tab_rows = prompt_table.reshape(NT * P, E)
    prow_idx = (task_ids.astype(jnp.int32)[:, None] * P
                + jnp.arange(P, dtype=jnp.int32)[None, :]).reshape(B * P)
    out = lax.cond(
        active,
        lambda a, b, c: _build_sc_copy(B, T, E, NT, P, False)(a, b, c),
        lambda a, b, c: _build_sc_copy(B, T, E, NT, P, True)(a, b, c),
        in_rows, tab_rows, prow_idx)
    return out.reshape(B, P + T, E)
